# baseline (speedup 1.0000x reference)
import functools

import jax
import jax.numpy as jnp
from jax import lax as _lax
from jax.experimental import pallas as pl
from jax.experimental.pallas import tpu as pltpu

_BN_EPS = 1e-5
_NUM_CLASS = 91
_K_SMALLEST = 45
_LANES = 128
_ROWS = 512
_ROWS3 = 2048
_N_COURSES = 8


def _bn_coeffs(s, ss, gamma, beta, inv_b):
    mean = s * inv_b
    var = ss * inv_b - mean * mean
    scale = gamma / jnp.sqrt(var + _BN_EPS)
    shift = beta - mean * scale
    return scale, shift


def _stage12_body(x_ref, w1_ref, b1_ref, g_ref, bt_ref, w2_ref, b2_ref,
                  h2_ref, s2_ref, ss2_ref, h1_ref, s1_ref, ss1_ref, *,
                  inv_b, nt):
    ph = pl.program_id(0)
    i = pl.program_id(1)

    @pl.when(ph == 0)
    def _():
        h = jnp.dot(x_ref[...], w1_ref[...],
                    preferred_element_type=jnp.float32)
        h = h + b1_ref[...]
        h1_ref[pl.ds(i * _ROWS, _ROWS), :] = h
        s = jnp.sum(h, axis=0, keepdims=True)
        ss = jnp.sum(h * h, axis=0, keepdims=True)

        @pl.when(i == 0)
        def _():
            s1_ref[...] = s
            ss1_ref[...] = ss

        @pl.when(i != 0)
        def _():
            s1_ref[...] += s
            ss1_ref[...] += ss

    @pl.when(ph == 1)
    def _():
        scale, shift = _bn_coeffs(s1_ref[...], ss1_ref[...], g_ref[...],
                                  bt_ref[...], inv_b)
        hn = jnp.maximum(h1_ref[pl.ds(i * _ROWS, _ROWS), :] * scale + shift,
                         0.0)
        h = jnp.dot(hn, w2_ref[...], preferred_element_type=jnp.float32)
        h = h + b2_ref[...]
        h2_ref[...] = h
        s = jnp.sum(h, axis=0, keepdims=True)
        ss = jnp.sum(h * h, axis=0, keepdims=True)

        @pl.when(i == 0)
        def _():
            s2_ref[...] = s
            ss2_ref[...] = ss

        @pl.when(i != 0)
        def _():
            s2_ref[...] += s
            ss2_ref[...] += ss


def _stage3_body(h_ref, s2_ref, ss2_ref, g_ref, bt_ref, w_ref, b_ref, t_ref,
                 out_ref, rt_ref, *, inv_b):
    f32 = jnp.float32
    scale, shift = _bn_coeffs(s2_ref[...], ss2_ref[...], g_ref[...],
                              bt_ref[...], inv_b)
    hn = jnp.maximum(h_ref[...] * scale + shift, 0.0)
    out = jnp.dot(hn, w_ref[...], preferred_element_type=f32) + b_ref[...]
    out_ref[...] = out

    lane = jax.lax.broadcasted_iota(jnp.int32, (_ROWS3, _LANES), 1)
    v = jnp.concatenate(
        [out, jnp.full((_ROWS3, _LANES - _NUM_CLASS), 1e30, f32)], axis=1)
    pr = jax.lax.broadcasted_iota(jnp.int32, (_LANES, _LANES), 0)
    pc = jax.lax.broadcasted_iota(jnp.int32, (_LANES, _LANES), 1)
    s = v
    k = 2
    while k <= _LANES:
        j = k // 2
        while j >= 1:
            perm = ((pr ^ j) == pc).astype(f32)
            w = jnp.dot(s, perm, preferred_element_type=f32)
            take_min = ((lane & j) == 0) == ((lane & k) == 0)
            s = jnp.where(take_min, jnp.minimum(s, w), jnp.maximum(s, w))
            j //= 2
        k *= 2
    ones = jnp.ones((_LANES, _LANES), f32)
    kf = f32(_K_SMALLEST)
    t_star = None
    for i in (_K_SMALLEST - 1, _K_SMALLEST - 2, _K_SMALLEST + 1,
              _K_SMALLEST):
        m = (s[:, i:i + 1] + s[:, i + 1:i + 2]) * f32(0.5)
        cm = jnp.dot(jnp.where(v < m, f32(1.0), f32(0.0)), ones,
                     preferred_element_type=f32)
        t_star = m if t_star is None else jnp.where(cm == kf, m, t_star)
    p = jnp.where(out < t_star[:, :_NUM_CLASS], f32(0.05), out)
    rt_ref[...] = jax.lax.dot_general(
        p, t_ref[...], (((1,), (1,)), ((), ())),
        preferred_element_type=f32)


def kernel(x, W1, b1, W2, b2, W4, b4, gamma, beta, topic_course):
    batch, nfeat = x.shape
    hidden = W1.shape[1]
    f32 = jnp.float32
    nt = batch // _ROWS
    inv_b = float(1.0 / batch)
    g = gamma.reshape(1, hidden)
    bt = beta.reshape(1, hidden)
    vec = lambda p, i: (0, 0)

    h2, s2, ss2 = pl.pallas_call(
        functools.partial(_stage12_body, inv_b=inv_b, nt=nt),
        grid=(2, nt),
        in_specs=[
            pl.BlockSpec((_ROWS, nfeat),
                         lambda p, i: (jnp.where(p == 0, i, 0), 0)),
            pl.BlockSpec((nfeat, hidden), vec),
            pl.BlockSpec((1, hidden), vec),
            pl.BlockSpec((1, hidden), vec),
            pl.BlockSpec((1, hidden), vec),
            pl.BlockSpec((hidden, hidden), vec),
            pl.BlockSpec((1, hidden), vec),
        ],
        out_specs=[
            pl.BlockSpec((_ROWS, hidden),
                         lambda p, i: (jnp.where(p == 1, i, 0), 0)),
            pl.BlockSpec((1, hidden), vec),
            pl.BlockSpec((1, hidden), vec),
        ],
        out_shape=[
            jax.ShapeDtypeStruct((batch, hidden), f32),
            jax.ShapeDtypeStruct((1, hidden), f32),
            jax.ShapeDtypeStruct((1, hidden), f32),
        ],
        scratch_shapes=[
            pltpu.VMEM((batch, hidden), f32),
            pltpu.VMEM((1, hidden), f32),
            pltpu.VMEM((1, hidden), f32),
        ],
    )(x, W1, b1.reshape(1, hidden), g, bt, W2, b2.reshape(1, hidden))

    outp, rtp = pl.pallas_call(
        functools.partial(_stage3_body, inv_b=inv_b),
        grid=(batch // _ROWS3,),
        in_specs=[
            pl.BlockSpec((_ROWS3, hidden), lambda i: (i, 0)),
            pl.BlockSpec((1, hidden), lambda i: (0, 0)),
            pl.BlockSpec((1, hidden), lambda i: (0, 0)),
            pl.BlockSpec((1, hidden), lambda i: (0, 0)),
            pl.BlockSpec((1, hidden), lambda i: (0, 0)),
            pl.BlockSpec((hidden, _NUM_CLASS), lambda i: (0, 0)),
            pl.BlockSpec((1, _NUM_CLASS), lambda i: (0, 0)),
            pl.BlockSpec((_N_COURSES, _NUM_CLASS), lambda i: (0, 0)),
        ],
        out_specs=[
            pl.BlockSpec((_ROWS3, _NUM_CLASS), lambda i: (i, 0)),
            pl.BlockSpec((_ROWS3, _N_COURSES), lambda i: (i, 0)),
        ],
        out_shape=[
            jax.ShapeDtypeStruct((batch, _NUM_CLASS), f32),
            jax.ShapeDtypeStruct((batch, _N_COURSES), f32),
        ],
    )(h2, s2, ss2, g, bt, W4, b4.reshape(1, _NUM_CLASS), topic_course)

    return (outp, rtp)


# speedup vs baseline: 1.0968x; 1.0968x over previous
import functools

import jax
import jax.numpy as jnp
from jax.experimental import pallas as pl
from jax.experimental.pallas import tpu as pltpu

_BN_EPS = 1e-5
_NUM_CLASS = 91
_K_SMALLEST = 45
_LANES = 128
_ROWS = 512
_ROWS3 = 2048
_N_COURSES = 8
_NT = 8
_NT3 = 2


def _bn_coeffs(s, ss, gamma, beta, inv_b):
    mean = s * inv_b
    var = ss * inv_b - mean * mean
    scale = gamma / jnp.sqrt(var + _BN_EPS)
    shift = beta - mean * scale
    return scale, shift


def _body(x_ref, w1_ref, b1_ref, g_ref, bt_ref, w2_ref, b2_ref, w4_ref,
          b4_ref, tc_ref, out_ref, rt_ref, h1_ref, h2_ref, s1_ref, ss1_ref,
          s2_ref, ss2_ref, *, inv_b):
    f32 = jnp.float32
    step = pl.program_id(0)

    @pl.when(step < _NT)
    def _stage1():
        i = step
        h = jnp.dot(x_ref[...], w1_ref[...], preferred_element_type=f32)
        h = h + b1_ref[...]
        h1_ref[pl.ds(i * _ROWS, _ROWS), :] = h
        s = jnp.sum(h, axis=0, keepdims=True)
        ss = jnp.sum(h * h, axis=0, keepdims=True)

        @pl.when(i == 0)
        def _():
            s1_ref[...] = s
            ss1_ref[...] = ss

        @pl.when(i != 0)
        def _():
            s1_ref[...] += s
            ss1_ref[...] += ss

    @pl.when((step >= _NT) & (step < 2 * _NT))
    def _stage2():
        i = step - _NT
        scale, shift = _bn_coeffs(s1_ref[...], ss1_ref[...], g_ref[...],
                                  bt_ref[...], inv_b)
        hn = jnp.maximum(h1_ref[pl.ds(i * _ROWS, _ROWS), :] * scale + shift,
                         0.0)
        h = jnp.dot(hn, w2_ref[...], preferred_element_type=f32)
        h = h + b2_ref[...]
        h2_ref[pl.ds(i * _ROWS, _ROWS), :] = h
        s = jnp.sum(h, axis=0, keepdims=True)
        ss = jnp.sum(h * h, axis=0, keepdims=True)

        @pl.when(i == 0)
        def _():
            s2_ref[...] = s
            ss2_ref[...] = ss

        @pl.when(i != 0)
        def _():
            s2_ref[...] += s
            ss2_ref[...] += ss

    @pl.when(step >= 2 * _NT)
    def _stage3():
        i = step - 2 * _NT
        scale, shift = _bn_coeffs(s2_ref[...], ss2_ref[...], g_ref[...],
                                  bt_ref[...], inv_b)
        hn = jnp.maximum(
            h2_ref[pl.ds(i * _ROWS3, _ROWS3), :] * scale + shift, 0.0)
        out = jnp.dot(hn, w4_ref[...], preferred_element_type=f32)
        out = out + b4_ref[...]
        out_ref[...] = out

        lane = jax.lax.broadcasted_iota(jnp.int32, (_ROWS3, _LANES), 1)
        v = jnp.concatenate(
            [out, jnp.full((_ROWS3, _LANES - _NUM_CLASS), 1e30, f32)],
            axis=1)
        pr = jax.lax.broadcasted_iota(jnp.int32, (_LANES, _LANES), 0)
        pc = jax.lax.broadcasted_iota(jnp.int32, (_LANES, _LANES), 1)
        s = v
        k = 2
        while k <= _LANES:
            j = k // 2
            while j >= 1:
                perm = ((pr ^ j) == pc).astype(f32)
                w = jnp.dot(s, perm, preferred_element_type=f32)
                take_min = ((lane & j) == 0) == ((lane & k) == 0)
                s = jnp.where(take_min, jnp.minimum(s, w), jnp.maximum(s, w))
                j //= 2
            k *= 2

        ones = jnp.ones((_LANES, _LANES), f32)
        kf = f32(_K_SMALLEST)
        t_star = None
        for i2 in (_K_SMALLEST - 1, _K_SMALLEST - 2, _K_SMALLEST + 1,
                   _K_SMALLEST):
            m = (s[:, i2:i2 + 1] + s[:, i2 + 1:i2 + 2]) * f32(0.5)
            cm = jnp.dot(jnp.where(v < m, f32(1.0), f32(0.0)), ones,
                         preferred_element_type=f32)
            t_star = m if t_star is None else jnp.where(cm == kf, m, t_star)
        p = jnp.where(out < t_star[:, :_NUM_CLASS], f32(0.05), out)
        rt_ref[...] = jax.lax.dot_general(
            p, tc_ref[...], (((1,), (1,)), ((), ())),
            preferred_element_type=f32)


def kernel(x, W1, b1, W2, b2, W4, b4, gamma, beta, topic_course):
    batch, nfeat = x.shape
    hidden = W1.shape[1]
    f32 = jnp.float32
    inv_b = float(1.0 / batch)
    g = gamma.reshape(1, hidden)
    bt = beta.reshape(1, hidden)
    vec = lambda s: (0, 0)

    outp, rtp = pl.pallas_call(
        functools.partial(_body, inv_b=inv_b),
        grid=(2 * _NT + _NT3,),
        in_specs=[
            pl.BlockSpec((_ROWS, nfeat),
                         lambda s: (jnp.minimum(s, _NT - 1), 0)),
            pl.BlockSpec((nfeat, hidden), vec),
            pl.BlockSpec((1, hidden), vec),
            pl.BlockSpec((1, hidden), vec),
            pl.BlockSpec((1, hidden), vec),
            pl.BlockSpec((hidden, hidden), vec),
            pl.BlockSpec((1, hidden), vec),
            pl.BlockSpec((hidden, _NUM_CLASS), vec),
            pl.BlockSpec((1, _NUM_CLASS), vec),
            pl.BlockSpec((_N_COURSES, _NUM_CLASS), vec),
        ],
        out_specs=[
            pl.BlockSpec((_ROWS3, _NUM_CLASS),
                         lambda s: (jnp.maximum(s - 2 * _NT, 0), 0)),
            pl.BlockSpec((_ROWS3, _N_COURSES),
                         lambda s: (jnp.maximum(s - 2 * _NT, 0), 0)),
        ],
        out_shape=[
            jax.ShapeDtypeStruct((batch, _NUM_CLASS), f32),
            jax.ShapeDtypeStruct((batch, _N_COURSES), f32),
        ],
        scratch_shapes=[
            pltpu.VMEM((batch, hidden), f32),
            pltpu.VMEM((batch, hidden), f32),
            pltpu.VMEM((1, hidden), f32),
            pltpu.VMEM((1, hidden), f32),
            pltpu.VMEM((1, hidden), f32),
            pltpu.VMEM((1, hidden), f32),
        ],
    )(x, W1, b1.reshape(1, hidden), g, bt, W2, b2.reshape(1, hidden),
      W4, b4.reshape(1, _NUM_CLASS), topic_course)

    return (outp, rtp)


# speedup vs baseline: 1.1684x; 1.0652x over previous
import functools

import jax
import jax.numpy as jnp
from jax.experimental import pallas as pl
from jax.experimental.pallas import tpu as pltpu

_BN_EPS = 1e-5
_NUM_CLASS = 91
_K_SMALLEST = 45
_LANES = 128
_ROWS = 1024
_ROWS3 = 2048
_N_COURSES = 8
_NT = 4
_NT3 = 2


def _bn_coeffs(s, ss, gamma, beta, inv_b):
    mean = s * inv_b
    var = ss * inv_b - mean * mean
    scale = gamma / jnp.sqrt(var + _BN_EPS)
    shift = beta - mean * scale
    return scale, shift


def _body(x_ref, w1_ref, b1_ref, g_ref, bt_ref, w2_ref, b2_ref, w4_ref,
          b4_ref, tc_ref, out_ref, rt_ref, h1_ref, h2_ref, s1_ref, ss1_ref,
          s2_ref, ss2_ref, *, inv_b):
    f32 = jnp.float32
    step = pl.program_id(0)

    @pl.when(step < _NT)
    def _stage1():
        i = step
        h = jnp.dot(x_ref[...], w1_ref[...], preferred_element_type=f32)
        h = h + b1_ref[...]
        h1_ref[pl.ds(i * _ROWS, _ROWS), :] = h
        s = jnp.sum(h, axis=0, keepdims=True)
        ss = jnp.sum(h * h, axis=0, keepdims=True)

        @pl.when(i == 0)
        def _():
            s1_ref[...] = s
            ss1_ref[...] = ss

        @pl.when(i != 0)
        def _():
            s1_ref[...] += s
            ss1_ref[...] += ss

    @pl.when((step >= _NT) & (step < 2 * _NT))
    def _stage2():
        i = step - _NT
        scale, shift = _bn_coeffs(s1_ref[...], ss1_ref[...], g_ref[...],
                                  bt_ref[...], inv_b)
        hn = jnp.maximum(h1_ref[pl.ds(i * _ROWS, _ROWS), :] * scale + shift,
                         0.0)
        h = jnp.dot(hn, w2_ref[...], preferred_element_type=f32)
        h = h + b2_ref[...]
        h2_ref[pl.ds(i * _ROWS, _ROWS), :] = h
        s = jnp.sum(h, axis=0, keepdims=True)
        ss = jnp.sum(h * h, axis=0, keepdims=True)

        @pl.when(i == 0)
        def _():
            s2_ref[...] = s
            ss2_ref[...] = ss

        @pl.when(i != 0)
        def _():
            s2_ref[...] += s
            ss2_ref[...] += ss

    @pl.when(step >= 2 * _NT)
    def _stage3():
        i = step - 2 * _NT
        scale, shift = _bn_coeffs(s2_ref[...], ss2_ref[...], g_ref[...],
                                  bt_ref[...], inv_b)
        hn = jnp.maximum(
            h2_ref[pl.ds(i * _ROWS3, _ROWS3), :] * scale + shift, 0.0)
        out = jnp.dot(hn, w4_ref[...], preferred_element_type=f32)
        out = out + b4_ref[...]
        out_ref[...] = out

        lane = jax.lax.broadcasted_iota(jnp.int32, (_ROWS3, _LANES), 1)
        v = jnp.concatenate(
            [out, jnp.full((_ROWS3, _LANES - _NUM_CLASS), 1e30, f32)],
            axis=1)
        pr = jax.lax.broadcasted_iota(jnp.int32, (_LANES, _LANES), 0)
        pc = jax.lax.broadcasted_iota(jnp.int32, (_LANES, _LANES), 1)
        s = v
        k = 2
        while k <= _LANES:
            j = k // 2
            while j >= 1:
                perm = ((pr ^ j) == pc).astype(f32)
                w = jnp.dot(s, perm, preferred_element_type=f32)
                take_min = ((lane & j) == 0) == ((lane & k) == 0)
                s = jnp.where(take_min, jnp.minimum(s, w), jnp.maximum(s, w))
                j //= 2
            k *= 2

        ones = jnp.ones((_LANES, _LANES), f32)
        kf = f32(_K_SMALLEST)
        t_star = None
        for i2 in (_K_SMALLEST - 1, _K_SMALLEST - 2, _K_SMALLEST + 1,
                   _K_SMALLEST):
            m = (s[:, i2:i2 + 1] + s[:, i2 + 1:i2 + 2]) * f32(0.5)
            cm = jnp.dot(jnp.where(v < m, f32(1.0), f32(0.0)), ones,
                         preferred_element_type=f32)
            t_star = m if t_star is None else jnp.where(cm == kf, m, t_star)
        p = jnp.where(out < t_star[:, :_NUM_CLASS], f32(0.05), out)
        rt_ref[...] = jax.lax.dot_general(
            p, tc_ref[...], (((1,), (1,)), ((), ())),
            preferred_element_type=f32)


def kernel(x, W1, b1, W2, b2, W4, b4, gamma, beta, topic_course):
    batch, nfeat = x.shape
    hidden = W1.shape[1]
    f32 = jnp.float32
    inv_b = float(1.0 / batch)
    g = gamma.reshape(1, hidden)
    bt = beta.reshape(1, hidden)
    vec = lambda s: (0, 0)

    outp, rtp = pl.pallas_call(
        functools.partial(_body, inv_b=inv_b),
        grid=(2 * _NT + _NT3,),
        in_specs=[
            pl.BlockSpec((_ROWS, nfeat),
                         lambda s: (jnp.minimum(s, _NT - 1), 0)),
            pl.BlockSpec((nfeat, hidden), vec),
            pl.BlockSpec((1, hidden), vec),
            pl.BlockSpec((1, hidden), vec),
            pl.BlockSpec((1, hidden), vec),
            pl.BlockSpec((hidden, hidden), vec),
            pl.BlockSpec((1, hidden), vec),
            pl.BlockSpec((hidden, _NUM_CLASS), vec),
            pl.BlockSpec((1, _NUM_CLASS), vec),
            pl.BlockSpec((_N_COURSES, _NUM_CLASS), vec),
        ],
        out_specs=[
            pl.BlockSpec((_ROWS3, _NUM_CLASS),
                         lambda s: (jnp.maximum(s - 2 * _NT, 0), 0)),
            pl.BlockSpec((_ROWS3, _N_COURSES),
                         lambda s: (jnp.maximum(s - 2 * _NT, 0), 0)),
        ],
        out_shape=[
            jax.ShapeDtypeStruct((batch, _NUM_CLASS), f32),
            jax.ShapeDtypeStruct((batch, _N_COURSES), f32),
        ],
        scratch_shapes=[
            pltpu.VMEM((batch, hidden), f32),
            pltpu.VMEM((batch, hidden), f32),
            pltpu.VMEM((1, hidden), f32),
            pltpu.VMEM((1, hidden), f32),
            pltpu.VMEM((1, hidden), f32),
            pltpu.VMEM((1, hidden), f32),
        ],
    )(x, W1, b1.reshape(1, hidden), g, bt, W2, b2.reshape(1, hidden),
      W4, b4.reshape(1, _NUM_CLASS), topic_course)

    return (outp, rtp)
